# baseline (speedup 1.0000x reference)
import functools

import jax
import jax.numpy as jnp
from jax import lax
from jax.experimental import pallas as pl
from jax.experimental.pallas import tpu as pltpu
from jax.experimental.pallas import tpu_sc as plsc

try:
    _INFO = plsc.get_sparse_core_info()
    _NC = _INFO.num_cores
    _NS = _INFO.num_subcores
except Exception:
    _NC, _NS = 2, 16
_NW = _NC * _NS

_CHUNK = 64
_NBUF = 15


def _make_gather(nb, ns, n_rows, d):
    n_idx = nb * ns
    assert n_idx % (_NW * _CHUNK) == 0
    per_w = n_idx // _NW
    n_chunks = per_w // _CHUNK
    nbuf = min(_NBUF, n_chunks)
    w_per_row = ns // per_w

    mesh = plsc.VectorSubcoreMesh(core_axis_name="c", subcore_axis_name="s")

    @functools.partial(
        pl.kernel,
        mesh=mesh,
        out_type=jax.ShapeDtypeStruct((n_idx, d), jnp.float32),
        scratch_types=(
            [pltpu.VMEM((per_w,), jnp.int32)]
            + [pltpu.VMEM((_CHUNK, d), jnp.float32) for _ in range(nbuf)]
            + [pltpu.SemaphoreType.DMA for _ in range(2 * nbuf)]
        ),
    )
    def gather_kernel(x_hbm, cache_hbm, out_hbm, idx_v, *rest):
        bufs = rest[:nbuf]
        gsem = rest[nbuf:2 * nbuf]
        ssem = rest[2 * nbuf:]
        wid = lax.axis_index("s") * _NC + lax.axis_index("c")
        base = wid * per_w

        brow = lax.div(wid, w_per_row)
        boff = lax.rem(wid, w_per_row) * per_w
        pltpu.sync_copy(x_hbm.at[brow, pl.ds(boff, per_w)], idx_v)

        gathers = [None] * n_chunks
        stores = [None] * n_chunks
        for j in range(min(nbuf, n_chunks)):
            gathers[j] = pltpu.async_copy(
                cache_hbm.at[idx_v.at[pl.ds(j * _CHUNK, _CHUNK)]],
                bufs[j], gsem[j])
        for j in range(n_chunks):
            b = j % nbuf
            gathers[j].wait()
            stores[j] = pltpu.async_copy(
                bufs[b], out_hbm.at[pl.ds(base + j * _CHUNK, _CHUNK)], ssem[b])
            jn = j + nbuf
            if jn < n_chunks:
                stores[j].wait()
                gathers[jn] = pltpu.async_copy(
                    cache_hbm.at[idx_v.at[pl.ds(jn * _CHUNK, _CHUNK)]],
                    bufs[b], gsem[b])
        for j in range(max(0, n_chunks - nbuf), n_chunks):
            if stores[j] is not None:
                stores[j].wait()

    return gather_kernel


def kernel(x, cache):
    b, s = x.shape
    n_rows, d = cache.shape
    out = _make_gather(b, s, n_rows, d)(x, cache)
    return out.reshape(b, s, d)


# speedup vs baseline: 1.0324x; 1.0324x over previous
import functools

import jax
import jax.numpy as jnp
from jax import lax
from jax.experimental import pallas as pl
from jax.experimental.pallas import tpu as pltpu
from jax.experimental.pallas import tpu_sc as plsc

try:
    _INFO = plsc.get_sparse_core_info()
    _NC = _INFO.num_cores
    _NS = _INFO.num_subcores
except Exception:
    _NC, _NS = 2, 16
_NW = _NC * _NS

_CHUNK = 128
_NBUF = 7


def _make_gather(nb, ns, n_rows, d):
    n_idx = nb * ns
    assert n_idx % (_NW * _CHUNK) == 0
    per_w = n_idx // _NW
    n_chunks = per_w // _CHUNK
    nbuf = min(_NBUF, n_chunks)
    w_per_row = ns // per_w

    mesh = plsc.VectorSubcoreMesh(core_axis_name="c", subcore_axis_name="s")

    @functools.partial(
        pl.kernel,
        mesh=mesh,
        out_type=jax.ShapeDtypeStruct((n_idx, d), jnp.float32),
        scratch_types=(
            [pltpu.VMEM((per_w,), jnp.int32)]
            + [pltpu.VMEM((_CHUNK, d), jnp.float32) for _ in range(nbuf)]
            + [pltpu.SemaphoreType.DMA for _ in range(2 * nbuf)]
        ),
    )
    def gather_kernel(x_hbm, cache_hbm, out_hbm, idx_v, *rest):
        bufs = rest[:nbuf]
        gsem = rest[nbuf:2 * nbuf]
        ssem = rest[2 * nbuf:]
        wid = lax.axis_index("s") * _NC + lax.axis_index("c")
        base = wid * per_w

        brow = lax.div(wid, w_per_row)
        boff = lax.rem(wid, w_per_row) * per_w
        pltpu.sync_copy(x_hbm.at[brow, pl.ds(boff, per_w)], idx_v)

        gathers = [None] * n_chunks
        stores = [None] * n_chunks
        for j in range(min(nbuf, n_chunks)):
            gathers[j] = pltpu.async_copy(
                cache_hbm.at[idx_v.at[pl.ds(j * _CHUNK, _CHUNK)]],
                bufs[j], gsem[j])
        for j in range(n_chunks):
            b = j % nbuf
            gathers[j].wait()
            stores[j] = pltpu.async_copy(
                bufs[b], out_hbm.at[pl.ds(base + j * _CHUNK, _CHUNK)], ssem[b])
            jn = j + nbuf
            if jn < n_chunks:
                stores[j].wait()
                gathers[jn] = pltpu.async_copy(
                    cache_hbm.at[idx_v.at[pl.ds(jn * _CHUNK, _CHUNK)]],
                    bufs[b], gsem[b])
        for j in range(max(0, n_chunks - nbuf), n_chunks):
            if stores[j] is not None:
                stores[j].wait()

    return gather_kernel


def kernel(x, cache):
    b, s = x.shape
    n_rows, d = cache.shape
    out = _make_gather(b, s, n_rows, d)(x, cache)
    return out.reshape(b, s, d)
